# baseline (speedup 1.0000x reference)
import functools

import jax
import jax.numpy as jnp
from jax import lax
from jax.experimental import pallas as pl
from jax.experimental.pallas import tpu as pltpu
from jax.experimental.pallas import tpu_sc as plsc

K = 100
V = 100000
D = 512
VB = 2048
T = (V + VB - 1) // VB

EPS = 1e-10


def _sum_body(phi_ref, s_ref):
    t = pl.program_id(0)
    blk = phi_ref[...]
    col = lax.broadcasted_iota(jnp.int32, blk.shape, 1) + t * VB
    blk = jnp.where(col < V, blk + 0.5, 0.0)

    @pl.when(t == 0)
    def _():
        s_ref[...] = jnp.zeros_like(s_ref)

    s_ref[...] += jnp.sum(blk, axis=1, keepdims=True)


def _phi_row_sums(phi):
    return pl.pallas_call(
        _sum_body,
        grid=(T,),
        in_specs=[pl.BlockSpec((K, VB), lambda t: (0, t))],
        out_specs=pl.BlockSpec((K, 1), lambda t: (0, 0)),
        out_shape=jax.ShapeDtypeStruct((K, 1), jnp.float32),
    )(phi)


def _main_body(theta_ref, s_ref, phi_ref, noise_ref,
               logp_ref, pval_ref, pidx_ref):
    t = pl.program_id(0)
    phin = (phi_ref[...] + 0.5) / s_ref[...]
    p = jnp.dot(theta_ref[...], phin, preferred_element_type=jnp.float32)
    logp = jnp.log(p + EPS)
    logp_ref[...] = logp
    u = noise_ref[...] * (1.0 - 2e-6) + 1e-6
    score = logp + (-jnp.log(-jnp.log(u)))
    col = lax.broadcasted_iota(jnp.int32, score.shape, 1) + t * VB
    score = jnp.where(col < V, score, -jnp.inf)
    m = jnp.max(score, axis=1, keepdims=True)
    cand = jnp.where(score == m, col, jnp.int32(2**31 - 1))
    idx = jnp.min(cand, axis=1, keepdims=True)
    pval_ref[...] = m[None]
    pidx_ref[...] = idx[None]


def _main_pass(theta_n, s, phi, noise):
    return pl.pallas_call(
        _main_body,
        grid=(T,),
        in_specs=[
            pl.BlockSpec((D, K), lambda t: (0, 0)),
            pl.BlockSpec((K, 1), lambda t: (0, 0)),
            pl.BlockSpec((K, VB), lambda t: (0, t)),
            pl.BlockSpec((D, VB), lambda t: (0, t)),
        ],
        out_specs=[
            pl.BlockSpec((D, VB), lambda t: (0, t)),
            pl.BlockSpec((1, D, 1), lambda t: (t, 0, 0)),
            pl.BlockSpec((1, D, 1), lambda t: (t, 0, 0)),
        ],
        out_shape=[
            jax.ShapeDtypeStruct((D, V), jnp.float32),
            jax.ShapeDtypeStruct((T, D, 1), jnp.float32),
            jax.ShapeDtypeStruct((T, D, 1), jnp.int32),
        ],
    )(theta_n, s, phi, noise)


def _merge_partials(pval, pidx):
    mesh = plsc.VectorSubcoreMesh(core_axis_name="c", subcore_axis_name="s")
    n_grp = D // 128

    @functools.partial(
        pl.kernel,
        mesh=mesh,
        out_type=jax.ShapeDtypeStruct((D,), jnp.int32),
        scratch_types=[
            pltpu.VMEM((T, 128), jnp.float32),
            pltpu.VMEM((T, 128), jnp.int32),
            pltpu.VMEM((128,), jnp.float32),
            pltpu.VMEM((128,), jnp.int32),
        ],
    )
    def merge(pval_hbm, pidx_hbm, w_hbm, val_v, idx_v, best_v, bidx_v):
        wid = lax.axis_index("c") * 16 + lax.axis_index("s")

        @pl.when(wid < n_grp)
        def _():
            base = wid * 128
            pltpu.sync_copy(pval_hbm.at[:, pl.ds(base, 128)], val_v)
            pltpu.sync_copy(pidx_hbm.at[:, pl.ds(base, 128)], idx_v)
            for c in range(8):
                sl = pl.ds(c * 16, 16)
                best_v[sl] = val_v[0, sl]
                bidx_v[sl] = idx_v[0, sl]

            @pl.loop(1, T)
            def _(t):
                for c in range(8):
                    sl = pl.ds(c * 16, 16)
                    v = val_v[t, sl]
                    upd = v > best_v[sl]
                    best_v[sl] = jnp.where(upd, v, best_v[sl])
                    bidx_v[sl] = jnp.where(upd, idx_v[t, sl], bidx_v[sl])

            pltpu.sync_copy(bidx_v, w_hbm.at[pl.ds(base, 128)])

    return merge(pval, pidx)


def kernel(phi_posterior, theta_posterior, uniform_noise):
    s = _phi_row_sums(phi_posterior)
    conc_theta = theta_posterior + 0.5
    theta_n = conc_theta / jnp.sum(conc_theta, axis=-1, keepdims=True)
    logp, pval, pidx = _main_pass(theta_n, s, phi_posterior, uniform_noise)
    w = _merge_partials(pval.reshape(T, D), pidx.reshape(T, D))
    return logp, w


# speedup vs baseline: 2.5274x; 2.5274x over previous
import functools

import jax
import jax.numpy as jnp
from jax import lax
from jax.experimental import pallas as pl
from jax.experimental.pallas import tpu as pltpu
from jax.experimental.pallas import tpu_sc as plsc

K = 100
V = 100000
D = 512
VB = 2048
T = (V + VB - 1) // VB

EPS = 1e-10


def _fused_body(theta_ref, phi_ref, noise_ref,
                logp_ref, pval_ref, pidx_ref, s_ref):
    i = pl.program_id(0)

    @pl.when(i == 0)
    def _():
        s_ref[...] = jnp.zeros_like(s_ref)

    @pl.when(i < T)
    def _():
        blk = phi_ref[...]
        col = lax.broadcasted_iota(jnp.int32, blk.shape, 1) + i * VB
        blk = jnp.where(col < V, blk + 0.5, 0.0)
        s_ref[...] += jnp.sum(blk, axis=1, keepdims=True)

    @pl.when(i >= T)
    def _():
        t = i - T
        phin = (phi_ref[...] + 0.5) * (1.0 / s_ref[...])
        p = lax.dot_general(phin, theta_ref[...],
                            (((0,), (0,)), ((), ())),
                            preferred_element_type=jnp.float32)
        logp = jnp.log(p + EPS)
        logp_ref[...] = logp
        u = noise_ref[...] * (1.0 - 2e-6) + 1e-6
        score = logp - jnp.log(-jnp.log(u))
        row = lax.broadcasted_iota(jnp.int32, score.shape, 0)

        def tail(sc):
            m = jnp.max(sc, axis=0, keepdims=True)
            cand = jnp.where(sc == m, row, jnp.int32(2**31 - 1))
            return m, jnp.min(cand, axis=0, keepdims=True)

        m, idx = lax.cond(
            t == T - 1,
            lambda: tail(jnp.where(row + t * VB < V, score, -jnp.inf)),
            lambda: tail(score),
        )
        pval_ref[...] = m[None]
        pidx_ref[...] = (idx + t * VB)[None]


def _main_pass(theta_nt, phi, noise_t):
    return pl.pallas_call(
        _fused_body,
        grid=(2 * T,),
        in_specs=[
            pl.BlockSpec((K, D), lambda i: (0, 0)),
            pl.BlockSpec((K, VB), lambda i: (0, i % T)),
            pl.BlockSpec((VB, D), lambda i: (jnp.maximum(i - T, 0), 0)),
        ],
        out_specs=[
            pl.BlockSpec((VB, D), lambda i: (jnp.maximum(i - T, 0), 0)),
            pl.BlockSpec((1, 1, D), lambda i: (jnp.maximum(i - T, 0), 0, 0)),
            pl.BlockSpec((1, 1, D), lambda i: (jnp.maximum(i - T, 0), 0, 0)),
        ],
        out_shape=[
            jax.ShapeDtypeStruct((V, D), jnp.float32),
            jax.ShapeDtypeStruct((T, 1, D), jnp.float32),
            jax.ShapeDtypeStruct((T, 1, D), jnp.int32),
        ],
        scratch_shapes=[pltpu.VMEM((K, 1), jnp.float32)],
        compiler_params=pltpu.CompilerParams(
            dimension_semantics=("arbitrary",)),
    )(theta_nt, phi, noise_t)


def _merge_partials(pval, pidx):
    mesh = plsc.VectorSubcoreMesh(core_axis_name="c", subcore_axis_name="s")
    n_grp = D // 128

    @functools.partial(
        pl.kernel,
        mesh=mesh,
        out_type=jax.ShapeDtypeStruct((D,), jnp.int32),
        scratch_types=[
            pltpu.VMEM((T, 128), jnp.float32),
            pltpu.VMEM((T, 128), jnp.int32),
            pltpu.VMEM((128,), jnp.float32),
            pltpu.VMEM((128,), jnp.int32),
        ],
    )
    def merge(pval_hbm, pidx_hbm, w_hbm, val_v, idx_v, best_v, bidx_v):
        wid = lax.axis_index("c") * 16 + lax.axis_index("s")

        @pl.when(wid < n_grp)
        def _():
            base = wid * 128
            pltpu.sync_copy(pval_hbm.at[:, pl.ds(base, 128)], val_v)
            pltpu.sync_copy(pidx_hbm.at[:, pl.ds(base, 128)], idx_v)
            for c in range(8):
                sl = pl.ds(c * 16, 16)
                best_v[sl] = val_v[0, sl]
                bidx_v[sl] = idx_v[0, sl]

            @pl.loop(1, T)
            def _(t):
                for c in range(8):
                    sl = pl.ds(c * 16, 16)
                    v = val_v[t, sl]
                    upd = v > best_v[sl]
                    best_v[sl] = jnp.where(upd, v, best_v[sl])
                    bidx_v[sl] = jnp.where(upd, idx_v[t, sl], bidx_v[sl])

            pltpu.sync_copy(bidx_v, w_hbm.at[pl.ds(base, 128)])

    return merge(pval, pidx)


def kernel(phi_posterior, theta_posterior, uniform_noise):
    conc_theta = theta_posterior + 0.5
    theta_n = conc_theta / jnp.sum(conc_theta, axis=-1, keepdims=True)
    logp_t, pval, pidx = _main_pass(theta_n.T, phi_posterior,
                                    uniform_noise.T)
    w = _merge_partials(pval.reshape(T, D), pidx.reshape(T, D))
    return logp_t.T, w


# speedup vs baseline: 2.6973x; 1.0672x over previous
import functools

import jax
import jax.numpy as jnp
from jax import lax
from jax.experimental import pallas as pl
from jax.experimental.pallas import tpu as pltpu
from jax.experimental.pallas import tpu_sc as plsc

K = 100
V = 100000
D = 512
VB = 2048
T = (V + VB - 1) // VB

EPS = 1e-10


VBS = 8192
TS = (V + VBS - 1) // VBS


def _sum_body(phi_ref, part_ref):
    t = pl.program_id(0)
    blk = phi_ref[...]
    col = lax.broadcasted_iota(jnp.int32, blk.shape, 1) + t * VBS
    blk = jnp.where(col < V, blk + 0.5, 0.0)
    part_ref[...] = jnp.sum(blk, axis=1, keepdims=True)[None]


def _phi_row_sums(phi):
    part = pl.pallas_call(
        _sum_body,
        grid=(TS,),
        in_specs=[pl.BlockSpec((K, VBS), lambda t: (0, t))],
        out_specs=pl.BlockSpec((1, K, 1), lambda t: (t, 0, 0)),
        out_shape=jax.ShapeDtypeStruct((TS, K, 1), jnp.float32),
        compiler_params=pltpu.CompilerParams(
            dimension_semantics=("parallel",)),
    )(phi)
    return jnp.sum(part, axis=0)


def _main_body(theta_ref, s_ref, phi_ref, noise_ref,
               logp_ref, pval_ref, pidx_ref):
    t = pl.program_id(0)
    phin = (phi_ref[...] + 0.5) * (1.0 / s_ref[...])
    p = lax.dot_general(phin, theta_ref[...],
                        (((0,), (0,)), ((), ())),
                        preferred_element_type=jnp.float32)
    logp = jnp.log(p + EPS)
    logp_ref[...] = logp
    u = noise_ref[...] * (1.0 - 2e-6) + 1e-6
    score = logp - jnp.log(-jnp.log(u))
    row = lax.broadcasted_iota(jnp.int32, score.shape, 0)

    def tail(sc):
        m = jnp.max(sc, axis=0, keepdims=True)
        cand = jnp.where(sc == m, row, jnp.int32(2**31 - 1))
        return m, jnp.min(cand, axis=0, keepdims=True)

    m, idx = lax.cond(
        t == T - 1,
        lambda: tail(jnp.where(row + t * VB < V, score, -jnp.inf)),
        lambda: tail(score),
    )
    pval_ref[...] = m[None]
    pidx_ref[...] = (idx + t * VB)[None]


def _main_pass(theta_nt, s, phi, noise_t):
    return pl.pallas_call(
        _main_body,
        grid=(T,),
        in_specs=[
            pl.BlockSpec((K, D), lambda t: (0, 0)),
            pl.BlockSpec((K, 1), lambda t: (0, 0)),
            pl.BlockSpec((K, VB), lambda t: (0, t)),
            pl.BlockSpec((VB, D), lambda t: (t, 0)),
        ],
        out_specs=[
            pl.BlockSpec((VB, D), lambda t: (t, 0)),
            pl.BlockSpec((1, 1, D), lambda t: (t, 0, 0)),
            pl.BlockSpec((1, 1, D), lambda t: (t, 0, 0)),
        ],
        out_shape=[
            jax.ShapeDtypeStruct((V, D), jnp.float32),
            jax.ShapeDtypeStruct((T, 1, D), jnp.float32),
            jax.ShapeDtypeStruct((T, 1, D), jnp.int32),
        ],
        compiler_params=pltpu.CompilerParams(
            dimension_semantics=("parallel",)),
    )(theta_nt, s, phi, noise_t)


def _merge_partials(pval, pidx):
    mesh = plsc.VectorSubcoreMesh(core_axis_name="c", subcore_axis_name="s")
    n_grp = D // 128

    @functools.partial(
        pl.kernel,
        mesh=mesh,
        out_type=jax.ShapeDtypeStruct((D,), jnp.int32),
        scratch_types=[
            pltpu.VMEM((T, 128), jnp.float32),
            pltpu.VMEM((T, 128), jnp.int32),
            pltpu.VMEM((128,), jnp.float32),
            pltpu.VMEM((128,), jnp.int32),
        ],
    )
    def merge(pval_hbm, pidx_hbm, w_hbm, val_v, idx_v, best_v, bidx_v):
        wid = lax.axis_index("c") * 16 + lax.axis_index("s")

        @pl.when(wid < n_grp)
        def _():
            base = wid * 128
            pltpu.sync_copy(pval_hbm.at[:, pl.ds(base, 128)], val_v)
            pltpu.sync_copy(pidx_hbm.at[:, pl.ds(base, 128)], idx_v)
            for c in range(8):
                sl = pl.ds(c * 16, 16)
                best_v[sl] = val_v[0, sl]
                bidx_v[sl] = idx_v[0, sl]

            @pl.loop(1, T)
            def _(t):
                for c in range(8):
                    sl = pl.ds(c * 16, 16)
                    v = val_v[t, sl]
                    upd = v > best_v[sl]
                    best_v[sl] = jnp.where(upd, v, best_v[sl])
                    bidx_v[sl] = jnp.where(upd, idx_v[t, sl], bidx_v[sl])

            pltpu.sync_copy(bidx_v, w_hbm.at[pl.ds(base, 128)])

    return merge(pval, pidx)


def kernel(phi_posterior, theta_posterior, uniform_noise):
    s = _phi_row_sums(phi_posterior)
    conc_theta = theta_posterior + 0.5
    theta_n = conc_theta / jnp.sum(conc_theta, axis=-1, keepdims=True)
    logp_t, pval, pidx = _main_pass(theta_n.T, s, phi_posterior,
                                    uniform_noise.T)
    w = _merge_partials(pval.reshape(T, D), pidx.reshape(T, D))
    return logp_t.T, w


# speedup vs baseline: 2.9304x; 1.0864x over previous
import functools

import jax
import jax.numpy as jnp
from jax import lax
from jax.experimental import pallas as pl
from jax.experimental.pallas import tpu as pltpu
from jax.experimental.pallas import tpu_sc as plsc

K = 100
V = 100000
D = 512
VB = 4096
T = (V + VB - 1) // VB

EPS = 1e-10


VBS = 8192
TS = (V + VBS - 1) // VBS


def _sum_body(phi_ref, part_ref):
    t = pl.program_id(0)
    blk = phi_ref[...]
    col = lax.broadcasted_iota(jnp.int32, blk.shape, 1) + t * VBS
    blk = jnp.where(col < V, blk + 0.5, 0.0)
    part_ref[...] = jnp.sum(blk, axis=1, keepdims=True)[None]


def _phi_row_sums(phi):
    part = pl.pallas_call(
        _sum_body,
        grid=(TS,),
        in_specs=[pl.BlockSpec((K, VBS), lambda t: (0, t))],
        out_specs=pl.BlockSpec((1, K, 1), lambda t: (t, 0, 0)),
        out_shape=jax.ShapeDtypeStruct((TS, K, 1), jnp.float32),
        compiler_params=pltpu.CompilerParams(
            dimension_semantics=("parallel",)),
    )(phi)
    return jnp.sum(part, axis=0)


def _main_body(theta_ref, s_ref, phi_ref, noise_ref,
               logp_ref, pval_ref, pidx_ref):
    t = pl.program_id(0)
    phin = (phi_ref[...] + 0.5) * (1.0 / s_ref[...])
    p = lax.dot_general(phin, theta_ref[...],
                        (((0,), (0,)), ((), ())),
                        preferred_element_type=jnp.float32)
    logp = jnp.log(p + EPS)
    logp_ref[...] = logp
    u = noise_ref[...] * (1.0 - 2e-6) + 1e-6
    score = logp - jnp.log(-jnp.log(u))
    row = lax.broadcasted_iota(jnp.int32, score.shape, 0)

    def tail(sc):
        m = jnp.max(sc, axis=0, keepdims=True)
        cand = jnp.where(sc == m, row, jnp.int32(2**31 - 1))
        return m, jnp.min(cand, axis=0, keepdims=True)

    m, idx = lax.cond(
        t == T - 1,
        lambda: tail(jnp.where(row + t * VB < V, score, -jnp.inf)),
        lambda: tail(score),
    )
    pval_ref[...] = m[None]
    pidx_ref[...] = (idx + t * VB)[None]


def _main_pass(theta_nt, s, phi, noise_t):
    return pl.pallas_call(
        _main_body,
        grid=(T,),
        in_specs=[
            pl.BlockSpec((K, D), lambda t: (0, 0)),
            pl.BlockSpec((K, 1), lambda t: (0, 0)),
            pl.BlockSpec((K, VB), lambda t: (0, t)),
            pl.BlockSpec((VB, D), lambda t: (t, 0)),
        ],
        out_specs=[
            pl.BlockSpec((VB, D), lambda t: (t, 0)),
            pl.BlockSpec((1, 1, D), lambda t: (t, 0, 0)),
            pl.BlockSpec((1, 1, D), lambda t: (t, 0, 0)),
        ],
        out_shape=[
            jax.ShapeDtypeStruct((V, D), jnp.float32),
            jax.ShapeDtypeStruct((T, 1, D), jnp.float32),
            jax.ShapeDtypeStruct((T, 1, D), jnp.int32),
        ],
        compiler_params=pltpu.CompilerParams(
            dimension_semantics=("parallel",)),
    )(theta_nt, s, phi, noise_t)


def _merge_partials(pval, pidx):
    mesh = plsc.VectorSubcoreMesh(core_axis_name="c", subcore_axis_name="s")
    n_grp = D // 128

    @functools.partial(
        pl.kernel,
        mesh=mesh,
        out_type=jax.ShapeDtypeStruct((D,), jnp.int32),
        scratch_types=[
            pltpu.VMEM((T, 128), jnp.float32),
            pltpu.VMEM((T, 128), jnp.int32),
            pltpu.VMEM((128,), jnp.float32),
            pltpu.VMEM((128,), jnp.int32),
        ],
    )
    def merge(pval_hbm, pidx_hbm, w_hbm, val_v, idx_v, best_v, bidx_v):
        wid = lax.axis_index("c") * 16 + lax.axis_index("s")

        @pl.when(wid < n_grp)
        def _():
            base = wid * 128
            pltpu.sync_copy(pval_hbm.at[:, pl.ds(base, 128)], val_v)
            pltpu.sync_copy(pidx_hbm.at[:, pl.ds(base, 128)], idx_v)
            for c in range(8):
                sl = pl.ds(c * 16, 16)
                best_v[sl] = val_v[0, sl]
                bidx_v[sl] = idx_v[0, sl]

            @pl.loop(1, T)
            def _(t):
                for c in range(8):
                    sl = pl.ds(c * 16, 16)
                    v = val_v[t, sl]
                    upd = v > best_v[sl]
                    best_v[sl] = jnp.where(upd, v, best_v[sl])
                    bidx_v[sl] = jnp.where(upd, idx_v[t, sl], bidx_v[sl])

            pltpu.sync_copy(bidx_v, w_hbm.at[pl.ds(base, 128)])

    return merge(pval, pidx)


def kernel(phi_posterior, theta_posterior, uniform_noise):
    s = _phi_row_sums(phi_posterior)
    conc_theta = theta_posterior + 0.5
    theta_n = conc_theta / jnp.sum(conc_theta, axis=-1, keepdims=True)
    logp_t, pval, pidx = _main_pass(theta_n.T, s, phi_posterior,
                                    uniform_noise.T)
    w = _merge_partials(pval.reshape(T, D), pidx.reshape(T, D))
    return logp_t.T, w
